# baseline (speedup 1.0000x reference)
import jax
import jax.numpy as jnp
from jax import lax
from jax.experimental import pallas as pl
from jax.experimental.pallas import tpu as pltpu
from jax.experimental.pallas import tpu_sc as plsc

B = 128
K = 8192
PS = 8
IMG = 64
NPATCH = 64
PD = 192
ED = 2048
VD = 2048

_AGENT = [r * 8 + c for r in range(3, 6) for c in range(3, 6)]
_CENTRAL = [r * 8 + c for r in range(2, 6) for c in range(2, 6)]

_M_BLK = 512
_B_BLK = _M_BLK // NPATCH


def _tc_idx_body(p_ref, cb_ref, out_ref, cnt_ref, cntag_ref):
    p = p_ref[...]
    s = jnp.sum(p * p, axis=1, keepdims=True)
    n = jnp.maximum(jnp.sqrt(s), 1e-8)
    a = (p / n).astype(jnp.bfloat16)
    sim = lax.dot_general(
        a, cb_ref[...], (((1,), (1,)), ((), ())),
        preferred_element_type=jnp.float32)
    idx = jnp.argmax(sim, axis=1).astype(jnp.int32)
    out_ref[...] = idx

    kio = lax.broadcasted_iota(jnp.int32, (_M_BLK, K), 1)
    oh = jnp.where(kio == idx[:, None], 1.0, 0.0).astype(jnp.bfloat16)
    rowb = lax.broadcasted_iota(jnp.int32, (_B_BLK, _M_BLK), 0)
    colp = lax.broadcasted_iota(jnp.int32, (_B_BLK, _M_BLK), 1)
    bd = (colp >> 6) == rowb
    pm = colp & 63
    ag = pm == _AGENT[0]
    for v in _AGENT[1:]:
        ag = ag | (pm == v)
    bd16 = jnp.where(bd, 1.0, 0.0).astype(jnp.bfloat16)
    bdag16 = jnp.where(bd & ag, 1.0, 0.0).astype(jnp.bfloat16)
    dims = (((1,), (0,)), ((), ()))
    cnt_ref[...] = lax.dot_general(
        bd16, oh, dims, preferred_element_type=jnp.float32)
    cntag_ref[...] = lax.dot_general(
        bdag16, oh, dims, preferred_element_type=jnp.float32)


def _tc_indices(patches, cb16):
    return pl.pallas_call(
        _tc_idx_body,
        grid=(B * NPATCH // _M_BLK,),
        in_specs=[
            pl.BlockSpec((_M_BLK, PD), lambda i: (i, 0)),
            pl.BlockSpec((K, PD), lambda i: (0, 0)),
        ],
        out_specs=[
            pl.BlockSpec((_M_BLK,), lambda i: (i,)),
            pl.BlockSpec((_B_BLK, K), lambda i: (i, 0)),
            pl.BlockSpec((_B_BLK, K), lambda i: (i, 0)),
        ],
        out_shape=[
            jax.ShapeDtypeStruct((B * NPATCH,), jnp.int32),
            jax.ShapeDtypeStruct((B, K), jnp.float32),
            jax.ShapeDtypeStruct((B, K), jnp.float32),
        ],
    )(patches, cb16)


_K_BLK = 2048
_N_STEP = K // _K_BLK


def _tc_bag_body(cnt_ref, cntag_ref, e_ref, zr_ref, zl_ref):
    i = pl.program_id(0)

    @pl.when(i == 0)
    def _init():
        zr_ref[...] = jnp.zeros((B, ED), jnp.float32)
        zl_ref[...] = jnp.zeros((B, ED), jnp.float32)

    e16 = e_ref[...].astype(jnp.bfloat16)
    c16 = cnt_ref[...].astype(jnp.bfloat16)
    ca16 = cntag_ref[...].astype(jnp.bfloat16)
    dims = (((1,), (0,)), ((), ()))
    zr_ref[...] += lax.dot_general(
        c16, e16, dims, preferred_element_type=jnp.float32)
    zl_ref[...] += lax.dot_general(
        ca16, e16, dims, preferred_element_type=jnp.float32)

    @pl.when(i == _N_STEP - 1)
    def _fin():
        zr_ref[...] = zr_ref[...] * (1.0 / 64.0)
        zl_ref[...] = zl_ref[...] * (1.0 / 9.0)


def _tc_embbag(cnt, cnt_ag, embeddings):
    return pl.pallas_call(
        _tc_bag_body,
        grid=(_N_STEP,),
        in_specs=[
            pl.BlockSpec((B, _K_BLK), lambda i: (0, i)),
            pl.BlockSpec((B, _K_BLK), lambda i: (0, i)),
            pl.BlockSpec((_K_BLK, ED), lambda i: (i, 0)),
        ],
        out_specs=[
            pl.BlockSpec((B, ED), lambda i: (0, 0)),
            pl.BlockSpec((B, ED), lambda i: (0, 0)),
        ],
        out_shape=[
            jax.ShapeDtypeStruct((B, ED), jnp.float32),
            jax.ShapeDtypeStruct((B, ED), jnp.float32),
        ],
    )(cnt, cnt_ag, embeddings)


_NC = 2
_NS = 16
_NW = _NC * _NS
_BPW = B // _NW
_NCHUNK = VD // 16


def _sc_vsa_body(idx_hbm, vsa_hbm, roles_hbm, cperm_hbm, zv_hbm,
                 cperm_v, pidx_v, gidx_v, buf_a, buf_b,
                 pos_v, zv_v, sem_a, sem_b, sem_i):
    wid = lax.axis_index("s") * _NC + lax.axis_index("c")

    pltpu.sync_copy(cperm_hbm, cperm_v)
    pltpu.async_copy(roles_hbm.at[cperm_v], pos_v, sem_i).wait()

    vbufs = [buf_a, buf_b]
    sems = [sem_a, sem_b]

    for k in range(_BPW):
        base = (wid * _BPW + k) * NPATCH
        pidx_v[pl.ds(k * 16, 16)] = cperm_v[...] + base
    gcp = pltpu.async_copy(idx_hbm.at[pidx_v], gidx_v, sem_i)

    def _tree(vals):
        vals = list(vals)
        while len(vals) > 1:
            nxt = [a + b for a, b in zip(vals[0::2], vals[1::2])]
            if len(vals) % 2:
                nxt.append(vals[-1])
            vals = nxt
        return vals[0]

    def vsa_fold(buf):
        def one(ds):
            t = []
            for r in range(16):
                d = buf[r, ds] - pos_v[r, ds]
                t.append(d * d)
            zv_v[ds] = jnp.where(_tree(t) > 8.0, 1.0, 0.0)

        def fold(c, _):
            one(pl.ds(c * 32, 16))
            one(pl.ds(c * 32 + 16, 16))
            return 0

        lax.fori_loop(0, _NCHUNK // 2, fold, 0)

    gcp.wait()
    cps = [None, None]
    cps[0] = pltpu.async_copy(vsa_hbm.at[gidx_v.at[pl.ds(0, 16)]],
                              buf_a, sem_a)
    cps[1] = pltpu.async_copy(vsa_hbm.at[gidx_v.at[pl.ds(16, 16)]],
                              buf_b, sem_b)
    for k in range(_BPW):
        if k + 2 < _BPW:
            nk = k % 2
            cps[nk].wait()
            vsa_fold(vbufs[nk])
            cps[nk] = pltpu.async_copy(
                vsa_hbm.at[gidx_v.at[pl.ds((k + 2) * 16, 16)]], vbufs[nk],
                sems[nk])
        else:
            cps[k % 2].wait()
            vsa_fold(vbufs[k % 2])
        pltpu.sync_copy(zv_v, zv_hbm.at[wid * _BPW + k])


def _sc_vsa(idx_flat, codebook_vsa, position_roles, cperm):
    f32 = jnp.float32
    i32 = jnp.int32
    mesh = plsc.VectorSubcoreMesh(
        core_axis_name="c", subcore_axis_name="s",
        num_cores=_NC, num_subcores=_NS)
    kfn = pl.kernel(
        _sc_vsa_body,
        out_type=jax.ShapeDtypeStruct((B, VD), f32),
        mesh=mesh,
        scratch_types=[
            pltpu.VMEM((16,), i32),
            pltpu.VMEM((NPATCH,), i32),
            pltpu.VMEM((NPATCH,), i32),
            pltpu.VMEM((16, VD), f32),
            pltpu.VMEM((16, VD), f32),
            pltpu.VMEM((16, VD), f32),
            pltpu.VMEM((VD,), f32),
            pltpu.SemaphoreType.DMA,
            pltpu.SemaphoreType.DMA,
            pltpu.SemaphoreType.DMA,
        ],
    )
    return kfn(idx_flat, codebook_vsa, position_roles, cperm)


def kernel(pixels, codebook, embeddings, codebook_vsa, position_roles):
    x = pixels.reshape(B, 3, IMG // PS, PS, IMG // PS, PS)
    x = jnp.transpose(x, (0, 2, 4, 1, 3, 5))
    patches = x.reshape(B * NPATCH, PD)
    cb16 = codebook.astype(jnp.bfloat16)
    idx_flat, cnt, cnt_ag = _tc_indices(patches, cb16)
    idx_mat = idx_flat.reshape(B, NPATCH)
    cperm = jnp.asarray(_CENTRAL, dtype=jnp.int32)
    z_vsa = _sc_vsa(idx_flat, codebook_vsa, position_roles, cperm)
    z_real, z_local = _tc_embbag(cnt, cnt_ag, embeddings)
    return z_real, z_vsa, idx_mat, z_local


# speedup vs baseline: 1.0858x; 1.0858x over previous
import jax
import jax.numpy as jnp
from jax import lax
from jax.experimental import pallas as pl
from jax.experimental.pallas import tpu as pltpu
from jax.experimental.pallas import tpu_sc as plsc

B = 128
K = 8192
PS = 8
IMG = 64
NPATCH = 64
PD = 192
ED = 2048
VD = 2048

_AGENT = [r * 8 + c for r in range(3, 6) for c in range(3, 6)]
_CENTRAL = [r * 8 + c for r in range(2, 6) for c in range(2, 6)]

_M_BLK = 256


def _tc_idx_body(p_ref, cb_ref, out_ref):
    p = p_ref[...]
    s = jnp.sum(p * p, axis=1, keepdims=True)
    n = jnp.maximum(jnp.sqrt(s), 1e-8)
    a = (p / n).astype(jnp.bfloat16)
    sim = lax.dot_general(
        a, cb_ref[...], (((1,), (1,)), ((), ())),
        preferred_element_type=jnp.float32)
    out_ref[...] = jnp.argmax(sim, axis=1).astype(jnp.int32)


def _tc_indices(patches, cb16):
    return pl.pallas_call(
        _tc_idx_body,
        grid=(B * NPATCH // _M_BLK,),
        in_specs=[
            pl.BlockSpec((_M_BLK, PD), lambda i: (i, 0)),
            pl.BlockSpec((K, PD), lambda i: (0, 0)),
        ],
        out_specs=pl.BlockSpec((_M_BLK,), lambda i: (i,)),
        out_shape=jax.ShapeDtypeStruct((B * NPATCH,), jnp.int32),
    )(patches, cb16)


_K_BLK = 2048
_K_SUB = 256
_N_STEP = K // _K_BLK


def _tc_bag_body(idxt_ref, e_ref, zr_ref, zl_ref, ct_ref, ca_ref):
    i = pl.program_id(0)

    @pl.when(i == 0)
    def _init():
        zr_ref[...] = jnp.zeros((B, ED), jnp.float32)
        zl_ref[...] = jnp.zeros((B, ED), jnp.float32)

    for j in range(_K_BLK // _K_SUB):
        kio = lax.broadcasted_iota(jnp.int32, (_K_SUB, B), 0) + (
            i * _K_BLK + j * _K_SUB)
        ct_ref[...] = jnp.zeros((_K_SUB, B), jnp.float32)

        def pbody(p, _, kio=kio):
            row = idxt_ref[pl.ds(p, 1), :]
            ct_ref[...] = ct_ref[...] + jnp.where(kio == row, 1.0, 0.0)
            return 0

        lax.fori_loop(0, NPATCH, pbody, 0)

        ca = jnp.where(kio == idxt_ref[pl.ds(_AGENT[0], 1), :], 1.0, 0.0)
        for p in _AGENT[1:]:
            ca = ca + jnp.where(kio == idxt_ref[pl.ds(p, 1), :], 1.0, 0.0)
        ca_ref[...] = ca

        e16 = e_ref[pl.ds(j * _K_SUB, _K_SUB), :].astype(jnp.bfloat16)
        ct16 = ct_ref[...].astype(jnp.bfloat16)
        ca16 = ca_ref[...].astype(jnp.bfloat16)
        dims = (((0,), (0,)), ((), ()))
        zr_ref[...] += lax.dot_general(
            ct16, e16, dims, preferred_element_type=jnp.float32)
        zl_ref[...] += lax.dot_general(
            ca16, e16, dims, preferred_element_type=jnp.float32)

    @pl.when(i == _N_STEP - 1)
    def _fin():
        zr_ref[...] = zr_ref[...] * (1.0 / 64.0)
        zl_ref[...] = zl_ref[...] * (1.0 / 9.0)


def _tc_embbag(idx_t, embeddings):
    return pl.pallas_call(
        _tc_bag_body,
        grid=(_N_STEP,),
        in_specs=[
            pl.BlockSpec((NPATCH, B), lambda i: (0, 0)),
            pl.BlockSpec((_K_BLK, ED), lambda i: (i, 0)),
        ],
        out_specs=[
            pl.BlockSpec((B, ED), lambda i: (0, 0)),
            pl.BlockSpec((B, ED), lambda i: (0, 0)),
        ],
        out_shape=[
            jax.ShapeDtypeStruct((B, ED), jnp.float32),
            jax.ShapeDtypeStruct((B, ED), jnp.float32),
        ],
        scratch_shapes=[
            pltpu.VMEM((_K_SUB, B), jnp.float32),
            pltpu.VMEM((_K_SUB, B), jnp.float32),
        ],
    )(idx_t, embeddings)


_NC = 2
_NS = 16
_NW = _NC * _NS
_BPW = B // _NW
_NCHUNK = VD // 16


def _sc_vsa_body(idx_hbm, vsa_hbm, roles_hbm, cperm_hbm, zv_hbm,
                 cperm_v, pidx_v, gidx0_v, gidx1_v, buf_a, buf_b,
                 pos_v, zv_v, sem_a, sem_b, sem_i):
    wid = lax.axis_index("s") * _NC + lax.axis_index("c")

    pltpu.sync_copy(cperm_hbm, cperm_v)
    pltpu.async_copy(roles_hbm.at[cperm_v], pos_v, sem_i).wait()

    gbufs = [gidx0_v, gidx1_v]
    vbufs = [buf_a, buf_b]
    sems = [sem_a, sem_b]

    def stage_gidx(k):
        base = (wid * _BPW + k) * NPATCH
        pidx_v[...] = cperm_v[...] + base
        return pltpu.async_copy(idx_hbm.at[pidx_v], gbufs[k % 2], sem_i)

    def _tree(vals):
        vals = list(vals)
        while len(vals) > 1:
            nxt = [a + b for a, b in zip(vals[0::2], vals[1::2])]
            if len(vals) % 2:
                nxt.append(vals[-1])
            vals = nxt
        return vals[0]

    def vsa_fold(buf):
        def one(ds):
            t = []
            for r in range(16):
                d = buf[r, ds] - pos_v[r, ds]
                t.append(d * d)
            zv_v[ds] = jnp.where(_tree(t) > 8.0, 1.0, 0.0)

        def fold(c, _):
            one(pl.ds(c * 32, 16))
            one(pl.ds(c * 32 + 16, 16))
            return 0

        lax.fori_loop(0, _NCHUNK // 2, fold, 0)

    stage_gidx(0).wait()
    cps = [None, None]
    cps[0] = pltpu.async_copy(vsa_hbm.at[gidx0_v], buf_a, sem_a)
    for k in range(_BPW):
        if k + 1 < _BPW:
            nk = (k + 1) % 2
            stage_gidx(k + 1).wait()
            cps[nk] = pltpu.async_copy(vsa_hbm.at[gbufs[nk]], vbufs[nk],
                                       sems[nk])
        cps[k % 2].wait()
        vsa_fold(vbufs[k % 2])
        pltpu.sync_copy(zv_v, zv_hbm.at[wid * _BPW + k])


def _sc_vsa(idx_flat, codebook_vsa, position_roles, cperm):
    f32 = jnp.float32
    i32 = jnp.int32
    mesh = plsc.VectorSubcoreMesh(
        core_axis_name="c", subcore_axis_name="s",
        num_cores=_NC, num_subcores=_NS)
    kfn = pl.kernel(
        _sc_vsa_body,
        out_type=jax.ShapeDtypeStruct((B, VD), f32),
        mesh=mesh,
        scratch_types=[
            pltpu.VMEM((16,), i32),
            pltpu.VMEM((16,), i32),
            pltpu.VMEM((16,), i32),
            pltpu.VMEM((16,), i32),
            pltpu.VMEM((16, VD), f32),
            pltpu.VMEM((16, VD), f32),
            pltpu.VMEM((16, VD), f32),
            pltpu.VMEM((VD,), f32),
            pltpu.SemaphoreType.DMA,
            pltpu.SemaphoreType.DMA,
            pltpu.SemaphoreType.DMA,
        ],
    )
    return kfn(idx_flat, codebook_vsa, position_roles, cperm)


def kernel(pixels, codebook, embeddings, codebook_vsa, position_roles):
    x = pixels.reshape(B, 3, IMG // PS, PS, IMG // PS, PS)
    x = jnp.transpose(x, (0, 2, 4, 1, 3, 5))
    patches = x.reshape(B * NPATCH, PD)
    cb16 = codebook.astype(jnp.bfloat16)
    idx_flat = _tc_indices(patches, cb16)
    idx_mat = idx_flat.reshape(B, NPATCH)
    idx_t = idx_mat.T
    z_real, z_local = _tc_embbag(idx_t, embeddings)
    cperm = jnp.asarray(_CENTRAL, dtype=jnp.int32)
    z_vsa = _sc_vsa(idx_flat, codebook_vsa, position_roles, cperm)
    return z_real, z_vsa, idx_mat, z_local
